# baseline (speedup 1.0000x reference)
import jax
import jax.numpy as jnp
from jax.experimental import pallas as pl
from jax.experimental.pallas import tpu as pltpu

_MAX_LEN = 2048
_BATCH = 16
_NUM_AA = 21
_ROWS = _NUM_AA * _BATCH
_AA_PER_BLOCK = 3
_BLOCK_R = _AA_PER_BLOCK * _BATCH


def _onehot_kernel(dataT_ref, len_ref, out_ref):
    dataT = dataT_ref[...]
    lens = len_ref[...]
    t = jax.lax.broadcasted_iota(jnp.int32, (_BATCH, _MAX_LEN), 1)
    q = jnp.where(t < lens, dataT, _NUM_AA)
    a0 = pl.program_id(0) * _AA_PER_BLOCK
    for k in range(_AA_PER_BLOCK):
        out_ref[k * _BATCH : (k + 1) * _BATCH, :] = (q == a0 + k).astype(
            jnp.float32
        )


def kernel(data, lengths, embed_init):
    del embed_init
    dataT = jnp.swapaxes(data, 0, 1)
    out_phys = pl.pallas_call(
        _onehot_kernel,
        grid=(_NUM_AA // _AA_PER_BLOCK,),
        in_specs=[
            pl.BlockSpec((_BATCH, _MAX_LEN), lambda i: (0, 0)),
            pl.BlockSpec((_BATCH, 1), lambda i: (0, 0)),
        ],
        out_specs=pl.BlockSpec((_BLOCK_R, _MAX_LEN), lambda i: (i, 0)),
        out_shape=jax.ShapeDtypeStruct((_ROWS, _MAX_LEN), jnp.float32),
        compiler_params=pltpu.CompilerParams(
            dimension_semantics=("parallel",)
        ),
    )(dataT, lengths.astype(jnp.int32).reshape(_BATCH, 1))
    return jnp.transpose(out_phys.reshape(_NUM_AA, _BATCH, _MAX_LEN), (2, 1, 0))


# speedup vs baseline: 1.3509x; 1.3509x over previous
import jax
import jax.numpy as jnp
from jax.experimental import pallas as pl
from jax.experimental.pallas import tpu as pltpu

_MAX_LEN = 2048
_BATCH = 16
_NUM_AA = 21
_ROWS = _NUM_AA * _BATCH
_AA_PER_BLOCK = 3
_BLOCK_R = _AA_PER_BLOCK * _BATCH


def _onehot_kernel(len_ref, dataT_ref, out_ref):
    dataT = dataT_ref[...]
    sub = jax.lax.broadcasted_iota(jnp.int32, (_BATCH, 1), 0)
    lens = jnp.full((_BATCH, 1), len_ref[0], jnp.int32)
    for b in range(1, _BATCH):
        lens = jnp.where(sub == b, len_ref[b], lens)
    t = jax.lax.broadcasted_iota(jnp.int32, (_BATCH, _MAX_LEN), 1)
    q = jnp.where(t < lens, dataT, _NUM_AA)
    a0 = pl.program_id(0) * _AA_PER_BLOCK
    for k in range(_AA_PER_BLOCK):
        out_ref[k * _BATCH : (k + 1) * _BATCH, :] = (q == a0 + k).astype(
            jnp.float32
        )


def kernel(data, lengths, embed_init):
    del embed_init
    dataT = jnp.swapaxes(data, 0, 1)
    out_phys = pl.pallas_call(
        _onehot_kernel,
        grid=(_NUM_AA // _AA_PER_BLOCK,),
        in_specs=[
            pl.BlockSpec(memory_space=pltpu.SMEM),
            pl.BlockSpec((_BATCH, _MAX_LEN), lambda i: (0, 0)),
        ],
        out_specs=pl.BlockSpec((_BLOCK_R, _MAX_LEN), lambda i: (i, 0)),
        out_shape=jax.ShapeDtypeStruct((_ROWS, _MAX_LEN), jnp.float32),
    )(lengths.astype(jnp.int32), dataT)
    return jnp.transpose(out_phys.reshape(_NUM_AA, _BATCH, _MAX_LEN), (2, 1, 0))


# speedup vs baseline: 1.8865x; 1.3964x over previous
import jax
import jax.numpy as jnp
from jax.experimental import pallas as pl
from jax.experimental.pallas import tpu as pltpu

_MAX_LEN = 2048
_BATCH = 16
_NUM_AA = 21
_ROWS = _NUM_AA * _BATCH
_AA_PER_BLOCK = 7
_BLOCK_R = _AA_PER_BLOCK * _BATCH


def _onehot_kernel(len_ref, dataT_ref, out_ref):
    dataT = dataT_ref[...]
    sub = jax.lax.broadcasted_iota(jnp.int32, (_BATCH, 1), 0)
    lens = jnp.full((_BATCH, 1), len_ref[0], jnp.int32)
    for b in range(1, _BATCH):
        lens = jnp.where(sub == b, len_ref[b], lens)
    t = jax.lax.broadcasted_iota(jnp.int32, (_BATCH, _MAX_LEN), 1)
    q = jnp.where(t < lens, dataT, _NUM_AA)
    a0 = pl.program_id(0) * _AA_PER_BLOCK
    for k in range(_AA_PER_BLOCK):
        out_ref[k * _BATCH : (k + 1) * _BATCH, :] = (q == a0 + k).astype(
            jnp.float32
        )


def kernel(data, lengths, embed_init):
    del embed_init
    dataT = jnp.swapaxes(data, 0, 1)
    out_phys = pl.pallas_call(
        _onehot_kernel,
        grid=(_NUM_AA // _AA_PER_BLOCK,),
        in_specs=[
            pl.BlockSpec(memory_space=pltpu.SMEM),
            pl.BlockSpec((_BATCH, _MAX_LEN), lambda i: (0, 0)),
        ],
        out_specs=pl.BlockSpec((_BLOCK_R, _MAX_LEN), lambda i: (i, 0)),
        out_shape=jax.ShapeDtypeStruct((_ROWS, _MAX_LEN), jnp.float32),
    )(lengths.astype(jnp.int32), dataT)
    return jnp.transpose(out_phys.reshape(_NUM_AA, _BATCH, _MAX_LEN), (2, 1, 0))


# speedup vs baseline: 2.1353x; 1.1319x over previous
import jax
import jax.numpy as jnp
from jax.experimental import pallas as pl
from jax.experimental.pallas import tpu as pltpu

_MAX_LEN = 2048
_BATCH = 16
_NUM_AA = 21
_ROWS = _NUM_AA * _BATCH
_AA_PER_BLOCK = 21
_BLOCK_R = _AA_PER_BLOCK * _BATCH


def _onehot_kernel(len_ref, dataT_ref, out_ref):
    dataT = dataT_ref[...]
    sub = jax.lax.broadcasted_iota(jnp.int32, (_BATCH, 1), 0)
    lens = jnp.full((_BATCH, 1), len_ref[0], jnp.int32)
    for b in range(1, _BATCH):
        lens = jnp.where(sub == b, len_ref[b], lens)
    t = jax.lax.broadcasted_iota(jnp.int32, (_BATCH, _MAX_LEN), 1)
    q = jnp.where(t < lens, dataT, _NUM_AA)
    a0 = pl.program_id(0) * _AA_PER_BLOCK
    for k in range(_AA_PER_BLOCK):
        out_ref[k * _BATCH : (k + 1) * _BATCH, :] = (q == a0 + k).astype(
            jnp.float32
        )


def kernel(data, lengths, embed_init):
    del embed_init
    dataT = jnp.swapaxes(data, 0, 1)
    out_phys = pl.pallas_call(
        _onehot_kernel,
        grid=(_NUM_AA // _AA_PER_BLOCK,),
        in_specs=[
            pl.BlockSpec(memory_space=pltpu.SMEM),
            pl.BlockSpec((_BATCH, _MAX_LEN), lambda i: (0, 0)),
        ],
        out_specs=pl.BlockSpec((_BLOCK_R, _MAX_LEN), lambda i: (i, 0)),
        out_shape=jax.ShapeDtypeStruct((_ROWS, _MAX_LEN), jnp.float32),
    )(lengths.astype(jnp.int32), dataT)
    return jnp.transpose(out_phys.reshape(_NUM_AA, _BATCH, _MAX_LEN), (2, 1, 0))
